# Initial kernel scaffold; baseline (speedup 1.0000x reference)
import jax
import jax.numpy as jnp
from jax.experimental import pallas as pl


def kernel(x, Ws, bs, edge_index_r0, edge_index_r1, edge_index_r2):
    raise NotImplementedError("write your pallas kernel here")


# speedup vs baseline: 3.0050x; 3.0050x over previous
import functools

import jax
import jax.numpy as jnp
from jax import lax
from jax.experimental import pallas as pl
from jax.experimental.pallas import tpu as pltpu
from jax.experimental.pallas import tpu_sc as plsc

N = 10000
D = 128
R = 3
L = 4
E = 160000

NC = 2
NS = 16
NW = NC * NS
CHUNK = 128
NCHUNKS = E // CHUNK
TRIPS_FLOOR = NCHUNKS // NW
TRIPS_EXTRA = NCHUNKS % NW
ROWS_PER_TILE = 624
ROWS_TAIL = N - NS * ROWS_PER_TILE
BN = 1000
NBLK = N // BN

_MESH = plsc.VectorSubcoreMesh(core_axis_name="c", subcore_axis_name="s")


def _sc_spmm(p0, p1, p2, s0, d0, s1, d1, s2, d2):
    @functools.partial(
        pl.kernel,
        out_type=jax.ShapeDtypeStruct((R, NC, N, D), jnp.float32),
        mesh=_MESH,
        scratch_types=[
            pltpu.VMEM((CHUNK,), jnp.int32),
            pltpu.VMEM((CHUNK,), jnp.int32),
            pltpu.VMEM((CHUNK, D), jnp.float32),
            pltpu.VMEM((16, D), jnp.float32),
            pltpu.VMEM_SHARED((N, D), jnp.float32),
            pltpu.SemaphoreType.DMA,
        ],
    )
    def k(p0_h, p1_h, p2_h, s0_h, d0_h, s1_h, d1_h, s2_h, d2_h, out_h,
          sidx, didx, rows, zbuf, acc, sem):
        core = lax.axis_index("c")
        sub = lax.axis_index("s")
        wid = core * NS + sub
        row0 = sub * ROWS_PER_TILE
        ntrips = jnp.where(wid < TRIPS_EXTRA, TRIPS_FLOOR + 1, TRIPS_FLOOR)

        z16 = jnp.zeros((16,), jnp.float32)

        @pl.loop(0, 16)
        def _(i):
            for j in range(D // 16):
                zbuf[i, pl.ds(j * 16, 16)] = z16

        for r, (p_h, s_h, d_h) in enumerate(
            [(p0_h, s0_h, d0_h), (p1_h, s1_h, d1_h), (p2_h, s2_h, d2_h)]
        ):
            @pl.loop(0, ROWS_PER_TILE // 16)
            def _(z):
                pltpu.sync_copy(zbuf, acc.at[pl.ds(row0 + z * 16, 16)])

            @pl.when(sub == NS - 1)
            def _():
                pltpu.sync_copy(zbuf, acc.at[pl.ds(N - ROWS_TAIL, ROWS_TAIL)])

            plsc.subcore_barrier()

            @pl.loop(0, ntrips)
            def _(j):
                off = (j * NW + wid) * CHUNK
                pltpu.sync_copy(s_h.at[pl.ds(off, CHUNK)], sidx)
                pltpu.sync_copy(d_h.at[pl.ds(off, CHUNK)], didx)
                pltpu.async_copy(p_h.at[sidx], rows, sem).wait()
                pltpu.sync_copy(rows, acc.at[didx], add=True)

            plsc.subcore_barrier()
            pltpu.sync_copy(acc.at[pl.ds(row0, ROWS_PER_TILE)],
                            out_h.at[r, core, pl.ds(row0, ROWS_PER_TILE)])

            @pl.when(sub == NS - 1)
            def _():
                pltpu.sync_copy(
                    acc.at[pl.ds(N - ROWS_TAIL, ROWS_TAIL)],
                    out_h.at[r, core, pl.ds(N - ROWS_TAIL, ROWS_TAIL)])

            plsc.subcore_barrier()

    return k(p0, p1, p2, s0, d0, s1, d1, s2, d2)


def _dot(a, b):
    return jax.lax.dot_general(
        a, b, (((1,), (0,)), ((), ())),
        precision=jax.lax.Precision.HIGHEST,
        preferred_element_type=jnp.float32,
    )


def _norm_from(degp_b, r):
    deg = degp_b[r, 0, :, 0] + degp_b[r, 1, :, 0]
    return jnp.where(deg > 0.0, lax.rsqrt(deg), 1.0)


def _tc_first(x, deg_out_p, deg_in_p, W0):
    def body(x_b, dop_b, dip_b, w_ref, p0_b, p1_b, p2_b, norm_b):
        ns = [_norm_from(dop_b, r) for r in range(R)]
        nd = [_norm_from(dip_b, r) for r in range(R)]
        one = jnp.ones((BN,), jnp.float32)
        norm_b[...] = jnp.stack(
            [ns[0], ns[1], ns[2], one, nd[0], nd[1], nd[2], one], axis=-1)
        xb = x_b[...]
        for r, p_b in enumerate([p0_b, p1_b, p2_b]):
            p_b[...] = ns[r][:, None] * _dot(xb, w_ref[r])

    out_shape = (
        [jax.ShapeDtypeStruct((N, D), jnp.float32)] * 3
        + [jax.ShapeDtypeStruct((N, 8), jnp.float32)]
    )
    part_spec = pl.BlockSpec((R, NC, BN, D), lambda i: (0, 0, i, 0))
    return pl.pallas_call(
        body,
        grid=(NBLK,),
        in_specs=[
            pl.BlockSpec((BN, D), lambda i: (i, 0)),
            part_spec,
            part_spec,
            pl.BlockSpec((R, D, D), lambda i: (0, 0, 0)),
        ],
        out_specs=[
            pl.BlockSpec((BN, D), lambda i: (i, 0)),
            pl.BlockSpec((BN, D), lambda i: (i, 0)),
            pl.BlockSpec((BN, D), lambda i: (i, 0)),
            pl.BlockSpec((BN, 8), lambda i: (i, 0)),
        ],
        out_shape=out_shape,
    )(x, deg_out_p, deg_in_p, W0)


def _tc_mid(ap, norms, Wl, b_prev, relu, matmul):
    def body(ap_b, norm_b, w_ref, b_ref, *outs):
        nrm = norm_b[...]
        h = jnp.zeros((BN, D), jnp.float32)
        for r in range(R):
            h = h + nrm[:, 4 + r][:, None] * (ap_b[r, 0] + ap_b[r, 1])
        h = h + (b_ref[0] + b_ref[1] + b_ref[2])[None, :]
        if relu:
            h = jnp.maximum(h, 0.0)
        if matmul:
            for r, p_b in enumerate(outs):
                p_b[...] = nrm[:, r][:, None] * _dot(h, w_ref[r])
        else:
            outs[0][...] = h

    n_out = 3 if matmul else 1
    return pl.pallas_call(
        body,
        grid=(NBLK,),
        in_specs=[
            pl.BlockSpec((R, NC, BN, D), lambda i: (0, 0, i, 0)),
            pl.BlockSpec((BN, 8), lambda i: (i, 0)),
            pl.BlockSpec((R, D, D), lambda i: (0, 0, 0)),
            pl.BlockSpec((R, D), lambda i: (0, 0)),
        ],
        out_specs=[pl.BlockSpec((BN, D), lambda i: (i, 0))] * n_out,
        out_shape=[jax.ShapeDtypeStruct((N, D), jnp.float32)] * n_out,
    )(ap, norms, Wl, b_prev)


def _tc_final_sum(ap):
    def body(ap_b, o_b):
        acc = jnp.zeros((BN, D), jnp.float32)
        for r in range(R):
            acc = acc + ap_b[r, 0] + ap_b[r, 1]
        o_b[...] = acc

    return pl.pallas_call(
        body,
        grid=(NBLK,),
        in_specs=[pl.BlockSpec((R, NC, BN, D), lambda i: (0, 0, i, 0))],
        out_specs=pl.BlockSpec((BN, D), lambda i: (i, 0)),
        out_shape=jax.ShapeDtypeStruct((N, D), jnp.float32),
    )(ap)


def kernel(x, Ws, bs, edge_index_r0, edge_index_r1, edge_index_r2):
    e0 = edge_index_r0.astype(jnp.int32)
    e1 = edge_index_r1.astype(jnp.int32)
    e2 = edge_index_r2.astype(jnp.int32)
    s0, d0 = e0[0], e0[1]
    s1, d1 = e1[0], e1[1]
    s2, d2 = e2[0], e2[1]

    ones_tbl = jnp.ones((N, D), jnp.float32)
    deg_out_p = _sc_spmm(ones_tbl, ones_tbl, ones_tbl, s0, s0, s1, s1, s2, s2)
    deg_in_p = _sc_spmm(ones_tbl, ones_tbl, ones_tbl, d0, d0, d1, d1, d2, d2)

    p0, p1, p2, norms = _tc_first(x, deg_out_p, deg_in_p, Ws[0])

    for layer in range(1, L):
        ap = _sc_spmm(p0, p1, p2, s0, d0, s1, d1, s2, d2)
        p0, p1, p2 = _tc_mid(ap, norms, Ws[layer], bs[layer - 1],
                             relu=True, matmul=True)

    ap = _sc_spmm(p0, p1, p2, s0, d0, s1, d1, s2, d2)
    hL = _tc_mid(ap, norms, Ws[0], bs[L - 1], relu=False, matmul=False)[0]

    ap2 = _sc_spmm(hL, hL, hL, s0, d0, s1, d1, s2, d2)
    return _tc_final_sum(ap2)


# speedup vs baseline: 4.7125x; 1.5682x over previous
import functools

import jax
import jax.numpy as jnp
from jax import lax
from jax.experimental import pallas as pl
from jax.experimental.pallas import tpu as pltpu
from jax.experimental.pallas import tpu_sc as plsc

N = 10000
D = 128
R = 3
L = 4
E = 160000

NC = 2
NS = 16
NW = NC * NS
CHUNK = 128
NCHUNKS = E // CHUNK
TRIPS_FLOOR = NCHUNKS // NW
TRIPS_EXTRA = NCHUNKS % NW
ROWS_PER_TILE = 624
ROWS_TAIL = N - NS * ROWS_PER_TILE
BN = 1000
NBLK = N // BN

_MESH = plsc.VectorSubcoreMesh(core_axis_name="c", subcore_axis_name="s")


def _edge_pass(p_h, s_h, d_h, acc, wid,
               sidxA, didxA, sidxB, didxB, rowsA, rowsB, gA, gB, iA, iB):
    T = TRIPS_FLOOR

    def idx_start(t, s_buf, d_buf, sem):
        off = (t * NW + wid) * CHUNK
        a = pltpu.make_async_copy(s_h.at[pl.ds(off, CHUNK)], s_buf, sem)
        b = pltpu.make_async_copy(d_h.at[pl.ds(off, CHUNK)], d_buf, sem)
        a.start()
        b.start()
        return a, b

    def idx_wait(s_buf, d_buf, sem):
        off = 0
        pltpu.make_async_copy(s_h.at[pl.ds(off, CHUNK)], s_buf, sem).wait()
        pltpu.make_async_copy(d_h.at[pl.ds(off, CHUNK)], d_buf, sem).wait()

    def gather_start(s_buf, rows, sem):
        pltpu.make_async_copy(p_h.at[s_buf], rows, sem).start()

    def gather_wait(s_buf, rows, sem):
        pltpu.make_async_copy(p_h.at[s_buf], rows, sem).wait()

    off0 = wid * CHUNK
    pltpu.sync_copy(s_h.at[pl.ds(off0, CHUNK)], sidxA)
    pltpu.sync_copy(d_h.at[pl.ds(off0, CHUNK)], didxA)
    gather_start(sidxA, rowsA, gA)
    gather_wait(sidxA, rowsA, gA)
    pltpu.sync_copy(rowsA, acc.at[didxA], add=True)

    off1 = (NW + wid) * CHUNK
    pltpu.sync_copy(s_h.at[pl.ds(off1, CHUNK)], sidxA)
    pltpu.sync_copy(d_h.at[pl.ds(off1, CHUNK)], didxA)
    gather_start(sidxA, rowsA, gA)
    idx_start(2, sidxB, didxB, iB)

    @pl.loop(0, (T - 1) // 2)
    def _(g):
        idx_wait(sidxB, didxB, iB)
        gather_start(sidxB, rowsB, gB)
        gather_wait(sidxA, rowsA, gA)
        pltpu.sync_copy(rowsA, acc.at[didxA], add=True)
        u = jnp.minimum(2 * g + 3, T - 1)
        idx_start(u, sidxA, didxA, iA)
        gather_wait(sidxB, rowsB, gB)
        pltpu.sync_copy(rowsB, acc.at[didxB], add=True)
        v = jnp.minimum(2 * g + 4, T - 1)
        idx_start(v, sidxB, didxB, iB)
        idx_wait(sidxA, didxA, iA)
        gather_start(sidxA, rowsA, gA)

    gather_wait(sidxA, rowsA, gA)
    idx_wait(sidxB, didxB, iB)

    @pl.when(wid < TRIPS_EXTRA)
    def _():
        offx = (T * NW + wid) * CHUNK
        pltpu.sync_copy(s_h.at[pl.ds(offx, CHUNK)], sidxA)
        pltpu.sync_copy(d_h.at[pl.ds(offx, CHUNK)], didxA)
        gather_start(sidxA, rowsA, gA)
        gather_wait(sidxA, rowsA, gA)
        pltpu.sync_copy(rowsA, acc.at[didxA], add=True)


def _sc_spmm(p0, p1, p2, s0, d0, s1, d1, s2, d2, shared=False):
    n_out = 1 if shared else R
    out_t = jax.ShapeDtypeStruct((n_out, NC, N, D), jnp.float32)

    @functools.partial(
        pl.kernel,
        out_type=out_t,
        mesh=_MESH,
        scratch_types=[
            pltpu.VMEM((CHUNK,), jnp.int32),
            pltpu.VMEM((CHUNK,), jnp.int32),
            pltpu.VMEM((CHUNK,), jnp.int32),
            pltpu.VMEM((CHUNK,), jnp.int32),
            pltpu.VMEM((CHUNK, D), jnp.float32),
            pltpu.VMEM((CHUNK, D), jnp.float32),
            pltpu.VMEM_SHARED((N, D), jnp.float32),
            pltpu.SemaphoreType.DMA,
            pltpu.SemaphoreType.DMA,
            pltpu.SemaphoreType.DMA,
            pltpu.SemaphoreType.DMA,
        ],
    )
    def k(z_h, p0_h, p1_h, p2_h, s0_h, d0_h, s1_h, d1_h, s2_h, d2_h, out_h,
          sidxA, didxA, sidxB, didxB, rowsA, rowsB, acc, gA, gB, iA, iB):
        core = lax.axis_index("c")
        sub = lax.axis_index("s")
        wid = core * NS + sub
        row0 = sub * ROWS_PER_TILE
        bufs = (sidxA, didxA, sidxB, didxB, rowsA, rowsB, gA, gB, iA, iB)

        def zero_acc():
            pltpu.sync_copy(z_h.at[pl.ds(row0, ROWS_PER_TILE)],
                            acc.at[pl.ds(row0, ROWS_PER_TILE)])

            @pl.when(sub == NS - 1)
            def _():
                pltpu.sync_copy(z_h.at[pl.ds(N - ROWS_TAIL, ROWS_TAIL)],
                                acc.at[pl.ds(N - ROWS_TAIL, ROWS_TAIL)])

        def drain(slot):
            pltpu.sync_copy(acc.at[pl.ds(row0, ROWS_PER_TILE)],
                            out_h.at[slot, core, pl.ds(row0, ROWS_PER_TILE)])

            @pl.when(sub == NS - 1)
            def _():
                pltpu.sync_copy(
                    acc.at[pl.ds(N - ROWS_TAIL, ROWS_TAIL)],
                    out_h.at[slot, core, pl.ds(N - ROWS_TAIL, ROWS_TAIL)])

        triples = [(p0_h, s0_h, d0_h), (p1_h, s1_h, d1_h), (p2_h, s2_h, d2_h)]
        if shared:
            zero_acc()
            plsc.subcore_barrier()
            for p_h, s_h, d_h in triples:
                _edge_pass(p_h, s_h, d_h, acc, wid, *bufs)
            plsc.subcore_barrier()
            drain(0)
        else:
            for r, (p_h, s_h, d_h) in enumerate(triples):
                zero_acc()
                plsc.subcore_barrier()
                _edge_pass(p_h, s_h, d_h, acc, wid, *bufs)
                plsc.subcore_barrier()
                drain(r)
                plsc.subcore_barrier()

    zeros = jnp.zeros((N, D), jnp.float32)
    return k(zeros, p0, p1, p2, s0, d0, s1, d1, s2, d2)


def _dot(a, b):
    return jax.lax.dot_general(
        a, b, (((1,), (0,)), ((), ())),
        precision=jax.lax.Precision.HIGHEST,
        preferred_element_type=jnp.float32,
    )


def _norm_from(degp_b, r):
    deg = degp_b[r, 0, :, 0] + degp_b[r, 1, :, 0]
    return jnp.where(deg > 0.0, lax.rsqrt(deg), 1.0)


def _tc_first(x, deg_out_p, deg_in_p, W0):
    def body(x_b, dop_b, dip_b, w_ref, p0_b, p1_b, p2_b, norm_b):
        ns = [_norm_from(dop_b, r) for r in range(R)]
        nd = [_norm_from(dip_b, r) for r in range(R)]
        one = jnp.ones((BN,), jnp.float32)
        norm_b[...] = jnp.stack(
            [ns[0], ns[1], ns[2], one, nd[0], nd[1], nd[2], one], axis=-1)
        xb = x_b[...]
        for r, p_b in enumerate([p0_b, p1_b, p2_b]):
            p_b[...] = ns[r][:, None] * _dot(xb, w_ref[r])

    out_shape = (
        [jax.ShapeDtypeStruct((N, D), jnp.float32)] * 3
        + [jax.ShapeDtypeStruct((N, 8), jnp.float32)]
    )
    part_spec = pl.BlockSpec((R, NC, BN, D), lambda i: (0, 0, i, 0))
    return pl.pallas_call(
        body,
        grid=(NBLK,),
        in_specs=[
            pl.BlockSpec((BN, D), lambda i: (i, 0)),
            part_spec,
            part_spec,
            pl.BlockSpec((R, D, D), lambda i: (0, 0, 0)),
        ],
        out_specs=[
            pl.BlockSpec((BN, D), lambda i: (i, 0)),
            pl.BlockSpec((BN, D), lambda i: (i, 0)),
            pl.BlockSpec((BN, D), lambda i: (i, 0)),
            pl.BlockSpec((BN, 8), lambda i: (i, 0)),
        ],
        out_shape=out_shape,
    )(x, deg_out_p, deg_in_p, W0)


def _tc_mid(ap, norms, Wl, b_prev, relu, matmul):
    def body(ap_b, norm_b, w_ref, b_ref, *outs):
        nrm = norm_b[...]
        h = jnp.zeros((BN, D), jnp.float32)
        for r in range(R):
            h = h + nrm[:, 4 + r][:, None] * (ap_b[r, 0] + ap_b[r, 1])
        h = h + (b_ref[0] + b_ref[1] + b_ref[2])[None, :]
        if relu:
            h = jnp.maximum(h, 0.0)
        if matmul:
            for r, p_b in enumerate(outs):
                p_b[...] = nrm[:, r][:, None] * _dot(h, w_ref[r])
        else:
            outs[0][...] = h

    n_out = 3 if matmul else 1
    return pl.pallas_call(
        body,
        grid=(NBLK,),
        in_specs=[
            pl.BlockSpec((R, NC, BN, D), lambda i: (0, 0, i, 0)),
            pl.BlockSpec((BN, 8), lambda i: (i, 0)),
            pl.BlockSpec((R, D, D), lambda i: (0, 0, 0)),
            pl.BlockSpec((R, D), lambda i: (0, 0)),
        ],
        out_specs=[pl.BlockSpec((BN, D), lambda i: (i, 0))] * n_out,
        out_shape=[jax.ShapeDtypeStruct((N, D), jnp.float32)] * n_out,
    )(ap, norms, Wl, b_prev)


def _tc_final_sum(ap):
    def body(ap_b, o_b):
        o_b[...] = ap_b[0, 0] + ap_b[0, 1]

    return pl.pallas_call(
        body,
        grid=(NBLK,),
        in_specs=[pl.BlockSpec((1, NC, BN, D), lambda i: (0, 0, i, 0))],
        out_specs=pl.BlockSpec((BN, D), lambda i: (i, 0)),
        out_shape=jax.ShapeDtypeStruct((N, D), jnp.float32),
    )(ap)


def kernel(x, Ws, bs, edge_index_r0, edge_index_r1, edge_index_r2):
    e0 = edge_index_r0.astype(jnp.int32)
    e1 = edge_index_r1.astype(jnp.int32)
    e2 = edge_index_r2.astype(jnp.int32)
    s0, d0 = e0[0], e0[1]
    s1, d1 = e1[0], e1[1]
    s2, d2 = e2[0], e2[1]

    ones_tbl = jnp.ones((N, D), jnp.float32)
    deg_out_p = _sc_spmm(ones_tbl, ones_tbl, ones_tbl, s0, s0, s1, s1, s2, s2)
    deg_in_p = _sc_spmm(ones_tbl, ones_tbl, ones_tbl, d0, d0, d1, d1, d2, d2)

    p0, p1, p2, norms = _tc_first(x, deg_out_p, deg_in_p, Ws[0])

    for layer in range(1, L):
        ap = _sc_spmm(p0, p1, p2, s0, d0, s1, d1, s2, d2)
        p0, p1, p2 = _tc_mid(ap, norms, Ws[layer], bs[layer - 1],
                             relu=True, matmul=True)

    ap = _sc_spmm(p0, p1, p2, s0, d0, s1, d1, s2, d2)
    hL = _tc_mid(ap, norms, Ws[0], bs[L - 1], relu=False, matmul=False)[0]

    ap2 = _sc_spmm(hL, hL, hL, s0, d0, s1, d1, s2, d2, shared=True)
    return _tc_final_sum(ap2)


# speedup vs baseline: 4.8907x; 1.0378x over previous
import functools

import jax
import jax.numpy as jnp
from jax import lax
from jax.experimental import pallas as pl
from jax.experimental.pallas import tpu as pltpu
from jax.experimental.pallas import tpu_sc as plsc

N = 10000
D = 128
R = 3
L = 4
E = 160000

NC = 2
NS = 16
NW = NC * NS
CHUNK = 128
NCHUNKS = E // CHUNK
TRIPS_FLOOR = NCHUNKS // NW
TRIPS_EXTRA = NCHUNKS % NW
ROWS_PER_TILE = 624
ROWS_TAIL = N - NS * ROWS_PER_TILE
BN = 1000
NBLK = N // BN

_MESH = plsc.VectorSubcoreMesh(core_axis_name="c", subcore_axis_name="s")


def _edge_pass(p_h, s_h, d_h, acc, wid,
               sidxA, didxA, sidxB, didxB, rowsA, rowsB,
               gA, gB, iA, iB, sA, sB):
    T = TRIPS_FLOOR

    def one_idx_start(t, e_h, buf, sem):
        off = (t * NW + wid) * CHUNK
        pltpu.make_async_copy(e_h.at[pl.ds(off, CHUNK)], buf, sem).start()

    def one_idx_wait(e_h, buf, sem):
        pltpu.make_async_copy(e_h.at[pl.ds(0, CHUNK)], buf, sem).wait()

    def gather_start(s_buf, rows, sem):
        pltpu.make_async_copy(p_h.at[s_buf], rows, sem).start()

    def gather_wait(s_buf, rows, sem):
        pltpu.make_async_copy(p_h.at[s_buf], rows, sem).wait()

    def scatter_start(rows, d_buf, sem):
        pltpu.make_async_copy(rows, acc.at[d_buf], sem).start(add=True)

    def scatter_wait(rows, d_buf, sem):
        pltpu.make_async_copy(rows, acc.at[d_buf], sem).wait()

    off0 = wid * CHUNK
    pltpu.sync_copy(s_h.at[pl.ds(off0, CHUNK)], sidxA)
    pltpu.sync_copy(d_h.at[pl.ds(off0, CHUNK)], didxA)
    gather_start(sidxA, rowsA, gA)
    gather_wait(sidxA, rowsA, gA)
    pltpu.sync_copy(rowsA, acc.at[didxA], add=True)

    off1 = (NW + wid) * CHUNK
    pltpu.sync_copy(s_h.at[pl.ds(off1, CHUNK)], sidxA)
    pltpu.sync_copy(d_h.at[pl.ds(off1, CHUNK)], didxA)
    gather_start(sidxA, rowsA, gA)
    one_idx_start(2, s_h, sidxB, iB)
    one_idx_start(2, d_h, didxB, iB)

    @pl.loop(0, (T - 1) // 2)
    def _(g):
        one_idx_wait(s_h, sidxB, iB)
        one_idx_wait(d_h, didxB, iB)
        gather_start(sidxB, rowsB, gB)
        gather_wait(sidxA, rowsA, gA)
        scatter_start(rowsA, didxA, sA)
        u = jnp.minimum(2 * g + 3, T - 1)
        one_idx_start(u, s_h, sidxA, iA)
        gather_wait(sidxB, rowsB, gB)
        scatter_start(rowsB, didxB, sB)
        scatter_wait(rowsA, didxA, sA)
        one_idx_start(u, d_h, didxA, iA)
        one_idx_wait(s_h, sidxA, iA)
        one_idx_wait(d_h, didxA, iA)
        gather_start(sidxA, rowsA, gA)
        scatter_wait(rowsB, didxB, sB)
        v = jnp.minimum(2 * g + 4, T - 1)
        one_idx_start(v, s_h, sidxB, iB)
        one_idx_start(v, d_h, didxB, iB)

    gather_wait(sidxA, rowsA, gA)
    one_idx_wait(s_h, sidxB, iB)
    one_idx_wait(d_h, didxB, iB)

    @pl.when(wid < TRIPS_EXTRA)
    def _():
        offx = (T * NW + wid) * CHUNK
        pltpu.sync_copy(s_h.at[pl.ds(offx, CHUNK)], sidxA)
        pltpu.sync_copy(d_h.at[pl.ds(offx, CHUNK)], didxA)
        gather_start(sidxA, rowsA, gA)
        gather_wait(sidxA, rowsA, gA)
        pltpu.sync_copy(rowsA, acc.at[didxA], add=True)


def _sc_spmm(p0, p1, p2, s0, d0, s1, d1, s2, d2, shared=False):
    n_out = 1 if shared else R
    out_t = jax.ShapeDtypeStruct((n_out, NC, N, D), jnp.float32)

    @functools.partial(
        pl.kernel,
        out_type=out_t,
        mesh=_MESH,
        scratch_types=[
            pltpu.VMEM((CHUNK,), jnp.int32),
            pltpu.VMEM((CHUNK,), jnp.int32),
            pltpu.VMEM((CHUNK,), jnp.int32),
            pltpu.VMEM((CHUNK,), jnp.int32),
            pltpu.VMEM((CHUNK, D), jnp.float32),
            pltpu.VMEM((CHUNK, D), jnp.float32),
            pltpu.VMEM_SHARED((N, D), jnp.float32),
            pltpu.SemaphoreType.DMA,
            pltpu.SemaphoreType.DMA,
            pltpu.SemaphoreType.DMA,
            pltpu.SemaphoreType.DMA,
            pltpu.SemaphoreType.DMA,
            pltpu.SemaphoreType.DMA,
        ],
    )
    def k(z_h, p0_h, p1_h, p2_h, s0_h, d0_h, s1_h, d1_h, s2_h, d2_h, out_h,
          sidxA, didxA, sidxB, didxB, rowsA, rowsB, acc,
          gA, gB, iA, iB, sA, sB):
        core = lax.axis_index("c")
        sub = lax.axis_index("s")
        wid = core * NS + sub
        row0 = sub * ROWS_PER_TILE
        bufs = (sidxA, didxA, sidxB, didxB, rowsA, rowsB,
                gA, gB, iA, iB, sA, sB)

        def zero_acc():
            pltpu.sync_copy(z_h.at[pl.ds(row0, ROWS_PER_TILE)],
                            acc.at[pl.ds(row0, ROWS_PER_TILE)])

            @pl.when(sub == NS - 1)
            def _():
                pltpu.sync_copy(z_h.at[pl.ds(N - ROWS_TAIL, ROWS_TAIL)],
                                acc.at[pl.ds(N - ROWS_TAIL, ROWS_TAIL)])

        def drain(slot):
            pltpu.sync_copy(acc.at[pl.ds(row0, ROWS_PER_TILE)],
                            out_h.at[slot, core, pl.ds(row0, ROWS_PER_TILE)])

            @pl.when(sub == NS - 1)
            def _():
                pltpu.sync_copy(
                    acc.at[pl.ds(N - ROWS_TAIL, ROWS_TAIL)],
                    out_h.at[slot, core, pl.ds(N - ROWS_TAIL, ROWS_TAIL)])

        triples = [(p0_h, s0_h, d0_h), (p1_h, s1_h, d1_h), (p2_h, s2_h, d2_h)]
        if shared:
            zero_acc()
            plsc.subcore_barrier()
            for p_h, s_h, d_h in triples:
                _edge_pass(p_h, s_h, d_h, acc, wid, *bufs)
            plsc.subcore_barrier()
            drain(0)
        else:
            for r, (p_h, s_h, d_h) in enumerate(triples):
                zero_acc()
                plsc.subcore_barrier()
                _edge_pass(p_h, s_h, d_h, acc, wid, *bufs)
                plsc.subcore_barrier()
                drain(r)
                plsc.subcore_barrier()

    zeros = jnp.zeros((N, D), jnp.float32)
    return k(zeros, p0, p1, p2, s0, d0, s1, d1, s2, d2)


def _dot(a, b):
    return jax.lax.dot_general(
        a, b, (((1,), (0,)), ((), ())),
        precision=jax.lax.Precision.HIGHEST,
        preferred_element_type=jnp.float32,
    )


def _norm_from(degp_b, r):
    deg = degp_b[r, 0, :, 0] + degp_b[r, 1, :, 0]
    return jnp.where(deg > 0.0, lax.rsqrt(deg), 1.0)


def _tc_first(x, deg_out_p, deg_in_p, W0):
    def body(x_b, dop_b, dip_b, w_ref, p0_b, p1_b, p2_b, norm_b):
        ns = [_norm_from(dop_b, r) for r in range(R)]
        nd = [_norm_from(dip_b, r) for r in range(R)]
        one = jnp.ones((BN,), jnp.float32)
        norm_b[...] = jnp.stack(
            [ns[0], ns[1], ns[2], one, nd[0], nd[1], nd[2], one], axis=-1)
        xb = x_b[...]
        for r, p_b in enumerate([p0_b, p1_b, p2_b]):
            p_b[...] = ns[r][:, None] * _dot(xb, w_ref[r])

    out_shape = (
        [jax.ShapeDtypeStruct((N, D), jnp.float32)] * 3
        + [jax.ShapeDtypeStruct((N, 8), jnp.float32)]
    )
    part_spec = pl.BlockSpec((R, NC, BN, D), lambda i: (0, 0, i, 0))
    return pl.pallas_call(
        body,
        grid=(NBLK,),
        in_specs=[
            pl.BlockSpec((BN, D), lambda i: (i, 0)),
            part_spec,
            part_spec,
            pl.BlockSpec((R, D, D), lambda i: (0, 0, 0)),
        ],
        out_specs=[
            pl.BlockSpec((BN, D), lambda i: (i, 0)),
            pl.BlockSpec((BN, D), lambda i: (i, 0)),
            pl.BlockSpec((BN, D), lambda i: (i, 0)),
            pl.BlockSpec((BN, 8), lambda i: (i, 0)),
        ],
        out_shape=out_shape,
    )(x, deg_out_p, deg_in_p, W0)


def _tc_mid(ap, norms, Wl, b_prev, relu, matmul):
    def body(ap_b, norm_b, w_ref, b_ref, *outs):
        nrm = norm_b[...]
        h = jnp.zeros((BN, D), jnp.float32)
        for r in range(R):
            h = h + nrm[:, 4 + r][:, None] * (ap_b[r, 0] + ap_b[r, 1])
        h = h + (b_ref[0] + b_ref[1] + b_ref[2])[None, :]
        if relu:
            h = jnp.maximum(h, 0.0)
        if matmul:
            for r, p_b in enumerate(outs):
                p_b[...] = nrm[:, r][:, None] * _dot(h, w_ref[r])
        else:
            outs[0][...] = h

    n_out = 3 if matmul else 1
    return pl.pallas_call(
        body,
        grid=(NBLK,),
        in_specs=[
            pl.BlockSpec((R, NC, BN, D), lambda i: (0, 0, i, 0)),
            pl.BlockSpec((BN, 8), lambda i: (i, 0)),
            pl.BlockSpec((R, D, D), lambda i: (0, 0, 0)),
            pl.BlockSpec((R, D), lambda i: (0, 0)),
        ],
        out_specs=[pl.BlockSpec((BN, D), lambda i: (i, 0))] * n_out,
        out_shape=[jax.ShapeDtypeStruct((N, D), jnp.float32)] * n_out,
    )(ap, norms, Wl, b_prev)


def _tc_final_sum(ap):
    def body(ap_b, o_b):
        o_b[...] = ap_b[0, 0] + ap_b[0, 1]

    return pl.pallas_call(
        body,
        grid=(NBLK,),
        in_specs=[pl.BlockSpec((1, NC, BN, D), lambda i: (0, 0, i, 0))],
        out_specs=pl.BlockSpec((BN, D), lambda i: (i, 0)),
        out_shape=jax.ShapeDtypeStruct((N, D), jnp.float32),
    )(ap)


def kernel(x, Ws, bs, edge_index_r0, edge_index_r1, edge_index_r2):
    e0 = edge_index_r0.astype(jnp.int32)
    e1 = edge_index_r1.astype(jnp.int32)
    e2 = edge_index_r2.astype(jnp.int32)
    s0, d0 = e0[0], e0[1]
    s1, d1 = e1[0], e1[1]
    s2, d2 = e2[0], e2[1]

    ones_tbl = jnp.ones((N, D), jnp.float32)
    deg_out_p = _sc_spmm(ones_tbl, ones_tbl, ones_tbl, s0, s0, s1, s1, s2, s2)
    deg_in_p = _sc_spmm(ones_tbl, ones_tbl, ones_tbl, d0, d0, d1, d1, d2, d2)

    p0, p1, p2, norms = _tc_first(x, deg_out_p, deg_in_p, Ws[0])

    for layer in range(1, L):
        ap = _sc_spmm(p0, p1, p2, s0, d0, s1, d1, s2, d2)
        p0, p1, p2 = _tc_mid(ap, norms, Ws[layer], bs[layer - 1],
                             relu=True, matmul=True)

    ap = _sc_spmm(p0, p1, p2, s0, d0, s1, d1, s2, d2)
    hL = _tc_mid(ap, norms, Ws[0], bs[L - 1], relu=False, matmul=False)[0]

    ap2 = _sc_spmm(hL, hL, hL, s0, d0, s1, d1, s2, d2, shared=True)
    return _tc_final_sum(ap2)


# speedup vs baseline: 5.4685x; 1.1182x over previous
import functools

import jax
import jax.numpy as jnp
from jax import lax
from jax.experimental import pallas as pl
from jax.experimental.pallas import tpu as pltpu
from jax.experimental.pallas import tpu_sc as plsc

N = 10000
D = 128
R = 3
L = 4
E = 160000

NC = 2
NS = 16
NW = NC * NS
CHUNK = 128
NCHUNKS = E // CHUNK
TRIPS_FLOOR = NCHUNKS // NW
TRIPS_EXTRA = NCHUNKS % NW
ROWS_PER_TILE = 624
ROWS_TAIL = N - NS * ROWS_PER_TILE
BN = 1000
NBLK = N // BN

_MESH = plsc.VectorSubcoreMesh(core_axis_name="c", subcore_axis_name="s")

def _deg_pass(e_h, acc, ones, wid, didxA, didxB, iA, iB, sA, sB):
    T = TRIPS_FLOOR

    def idx_start(t, buf, sem):
        off = (t * NW + wid) * CHUNK
        pltpu.make_async_copy(e_h.at[pl.ds(off, CHUNK)], buf, sem).start()

    def idx_wait(buf, sem):
        pltpu.make_async_copy(e_h.at[pl.ds(0, CHUNK)], buf, sem).wait()

    def scatter_start(d_buf, sem):
        pltpu.make_async_copy(ones, acc.at[d_buf], sem).start(add=True)

    def scatter_wait(d_buf, sem):
        pltpu.make_async_copy(ones, acc.at[d_buf], sem).wait()

    pltpu.sync_copy(e_h.at[pl.ds(wid * CHUNK, CHUNK)], didxA)
    pltpu.sync_copy(ones, acc.at[didxA], add=True)
    pltpu.sync_copy(e_h.at[pl.ds((NW + wid) * CHUNK, CHUNK)], didxA)
    idx_start(2, didxB, iB)

    @pl.loop(0, (T - 1) // 2)
    def _(g):
        scatter_start(didxA, sA)
        idx_wait(didxB, iB)
        scatter_start(didxB, sB)
        scatter_wait(didxA, sA)
        idx_start(jnp.minimum(2 * g + 3, T - 1), didxA, iA)
        scatter_wait(didxB, sB)
        idx_start(jnp.minimum(2 * g + 4, T - 1), didxB, iB)
        idx_wait(didxA, iA)

    idx_wait(didxB, iB)

    @pl.when(wid < TRIPS_EXTRA)
    def _():
        pltpu.sync_copy(e_h.at[pl.ds((T * NW + wid) * CHUNK, CHUNK)], didxA)
        pltpu.sync_copy(ones, acc.at[didxA], add=True)


def _sc_degrees(s0, s1, s2, d0, d1, d2):
    zeros = jnp.zeros((N, D), jnp.float32)

    @functools.partial(
        pl.kernel,
        out_type=jax.ShapeDtypeStruct((6, NC, N, D), jnp.float32),
        mesh=_MESH,
        scratch_types=[
            pltpu.VMEM((CHUNK,), jnp.int32),
            pltpu.VMEM((CHUNK,), jnp.int32),
            pltpu.VMEM((CHUNK, D), jnp.float32),
            pltpu.VMEM_SHARED((N, D), jnp.float32),
            pltpu.SemaphoreType.DMA,
            pltpu.SemaphoreType.DMA,
            pltpu.SemaphoreType.DMA,
            pltpu.SemaphoreType.DMA,
        ],
    )
    def k(z_h, e0_h, e1_h, e2_h, e3_h, e4_h, e5_h, out_h,
          didxA, didxB, ones, acc, iA, iB, sA, sB):
        core = lax.axis_index("c")
        sub = lax.axis_index("s")
        wid = core * NS + sub
        row0 = sub * ROWS_PER_TILE

        one = jnp.ones((16,), jnp.float32)

        @pl.loop(0, CHUNK)
        def _(i):
            for j in range(D // 16):
                ones[i, pl.ds(j * 16, 16)] = one

        for slot, e_h in enumerate([e0_h, e1_h, e2_h, e3_h, e4_h, e5_h]):
            pltpu.sync_copy(z_h.at[pl.ds(row0, ROWS_PER_TILE)],
                            acc.at[pl.ds(row0, ROWS_PER_TILE)])

            @pl.when(sub == NS - 1)
            def _():
                pltpu.sync_copy(z_h.at[pl.ds(N - ROWS_TAIL, ROWS_TAIL)],
                                acc.at[pl.ds(N - ROWS_TAIL, ROWS_TAIL)])

            plsc.subcore_barrier()
            _deg_pass(e_h, acc, ones, wid, didxA, didxB, iA, iB, sA, sB)
            plsc.subcore_barrier()
            pltpu.sync_copy(acc.at[pl.ds(row0, ROWS_PER_TILE)],
                            out_h.at[slot, core, pl.ds(row0, ROWS_PER_TILE)])

            @pl.when(sub == NS - 1)
            def _():
                pltpu.sync_copy(
                    acc.at[pl.ds(N - ROWS_TAIL, ROWS_TAIL)],
                    out_h.at[slot, core, pl.ds(N - ROWS_TAIL, ROWS_TAIL)])

            plsc.subcore_barrier()

    return k(zeros, s0, s1, s2, d0, d1, d2)


def _edge_pass(p_h, s_h, d_h, acc, wid,
               sidxA, didxA, sidxB, didxB, rowsA, rowsB,
               gA, gB, iA, iB, sA, sB):
    T = TRIPS_FLOOR

    def one_idx_start(t, e_h, buf, sem):
        off = (t * NW + wid) * CHUNK
        pltpu.make_async_copy(e_h.at[pl.ds(off, CHUNK)], buf, sem).start()

    def one_idx_wait(e_h, buf, sem):
        pltpu.make_async_copy(e_h.at[pl.ds(0, CHUNK)], buf, sem).wait()

    def gather_start(s_buf, rows, sem):
        pltpu.make_async_copy(p_h.at[s_buf], rows, sem).start()

    def gather_wait(s_buf, rows, sem):
        pltpu.make_async_copy(p_h.at[s_buf], rows, sem).wait()

    def scatter_start(rows, d_buf, sem):
        pltpu.make_async_copy(rows, acc.at[d_buf], sem).start(add=True)

    def scatter_wait(rows, d_buf, sem):
        pltpu.make_async_copy(rows, acc.at[d_buf], sem).wait()

    off0 = wid * CHUNK
    pltpu.sync_copy(s_h.at[pl.ds(off0, CHUNK)], sidxA)
    pltpu.sync_copy(d_h.at[pl.ds(off0, CHUNK)], didxA)
    gather_start(sidxA, rowsA, gA)
    gather_wait(sidxA, rowsA, gA)
    pltpu.sync_copy(rowsA, acc.at[didxA], add=True)

    off1 = (NW + wid) * CHUNK
    pltpu.sync_copy(s_h.at[pl.ds(off1, CHUNK)], sidxA)
    pltpu.sync_copy(d_h.at[pl.ds(off1, CHUNK)], didxA)
    gather_start(sidxA, rowsA, gA)
    one_idx_start(2, s_h, sidxB, iB)
    one_idx_start(2, d_h, didxB, iB)

    @pl.loop(0, (T - 1) // 2)
    def _(g):
        one_idx_wait(s_h, sidxB, iB)
        one_idx_wait(d_h, didxB, iB)
        gather_start(sidxB, rowsB, gB)
        gather_wait(sidxA, rowsA, gA)
        scatter_start(rowsA, didxA, sA)
        u = jnp.minimum(2 * g + 3, T - 1)
        one_idx_start(u, s_h, sidxA, iA)
        gather_wait(sidxB, rowsB, gB)
        scatter_start(rowsB, didxB, sB)
        scatter_wait(rowsA, didxA, sA)
        one_idx_start(u, d_h, didxA, iA)
        one_idx_wait(s_h, sidxA, iA)
        one_idx_wait(d_h, didxA, iA)
        gather_start(sidxA, rowsA, gA)
        scatter_wait(rowsB, didxB, sB)
        v = jnp.minimum(2 * g + 4, T - 1)
        one_idx_start(v, s_h, sidxB, iB)
        one_idx_start(v, d_h, didxB, iB)

    gather_wait(sidxA, rowsA, gA)
    one_idx_wait(s_h, sidxB, iB)
    one_idx_wait(d_h, didxB, iB)

    @pl.when(wid < TRIPS_EXTRA)
    def _():
        offx = (T * NW + wid) * CHUNK
        pltpu.sync_copy(s_h.at[pl.ds(offx, CHUNK)], sidxA)
        pltpu.sync_copy(d_h.at[pl.ds(offx, CHUNK)], didxA)
        gather_start(sidxA, rowsA, gA)
        gather_wait(sidxA, rowsA, gA)
        pltpu.sync_copy(rowsA, acc.at[didxA], add=True)


def _sc_spmm(p0, p1, p2, s0, d0, s1, d1, s2, d2, shared=False):
    n_out = 1 if shared else R
    out_t = jax.ShapeDtypeStruct((n_out, NC, N, D), jnp.float32)

    @functools.partial(
        pl.kernel,
        out_type=out_t,
        mesh=_MESH,
        scratch_types=[
            pltpu.VMEM((CHUNK,), jnp.int32),
            pltpu.VMEM((CHUNK,), jnp.int32),
            pltpu.VMEM((CHUNK,), jnp.int32),
            pltpu.VMEM((CHUNK,), jnp.int32),
            pltpu.VMEM((CHUNK, D), jnp.float32),
            pltpu.VMEM((CHUNK, D), jnp.float32),
            pltpu.VMEM_SHARED((N, D), jnp.float32),
            pltpu.SemaphoreType.DMA,
            pltpu.SemaphoreType.DMA,
            pltpu.SemaphoreType.DMA,
            pltpu.SemaphoreType.DMA,
            pltpu.SemaphoreType.DMA,
            pltpu.SemaphoreType.DMA,
        ],
    )
    def k(z_h, p0_h, p1_h, p2_h, s0_h, d0_h, s1_h, d1_h, s2_h, d2_h, out_h,
          sidxA, didxA, sidxB, didxB, rowsA, rowsB, acc,
          gA, gB, iA, iB, sA, sB):
        core = lax.axis_index("c")
        sub = lax.axis_index("s")
        wid = core * NS + sub
        row0 = sub * ROWS_PER_TILE
        bufs = (sidxA, didxA, sidxB, didxB, rowsA, rowsB,
                gA, gB, iA, iB, sA, sB)

        def zero_acc():
            pltpu.sync_copy(z_h.at[pl.ds(row0, ROWS_PER_TILE)],
                            acc.at[pl.ds(row0, ROWS_PER_TILE)])

            @pl.when(sub == NS - 1)
            def _():
                pltpu.sync_copy(z_h.at[pl.ds(N - ROWS_TAIL, ROWS_TAIL)],
                                acc.at[pl.ds(N - ROWS_TAIL, ROWS_TAIL)])

        def drain(slot):
            pltpu.sync_copy(acc.at[pl.ds(row0, ROWS_PER_TILE)],
                            out_h.at[slot, core, pl.ds(row0, ROWS_PER_TILE)])

            @pl.when(sub == NS - 1)
            def _():
                pltpu.sync_copy(
                    acc.at[pl.ds(N - ROWS_TAIL, ROWS_TAIL)],
                    out_h.at[slot, core, pl.ds(N - ROWS_TAIL, ROWS_TAIL)])

        triples = [(p0_h, s0_h, d0_h), (p1_h, s1_h, d1_h), (p2_h, s2_h, d2_h)]
        if shared:
            zero_acc()
            plsc.subcore_barrier()
            for p_h, s_h, d_h in triples:
                _edge_pass(p_h, s_h, d_h, acc, wid, *bufs)
            plsc.subcore_barrier()
            drain(0)
        else:
            for r, (p_h, s_h, d_h) in enumerate(triples):
                zero_acc()
                plsc.subcore_barrier()
                _edge_pass(p_h, s_h, d_h, acc, wid, *bufs)
                plsc.subcore_barrier()
                drain(r)
                plsc.subcore_barrier()

    zeros = jnp.zeros((N, D), jnp.float32)
    return k(zeros, p0, p1, p2, s0, d0, s1, d1, s2, d2)


def _dot(a, b):
    return jax.lax.dot_general(
        a, b, (((1,), (0,)), ((), ())),
        precision=jax.lax.Precision.HIGHEST,
        preferred_element_type=jnp.float32,
    )


def _norm_from(degp_b, slot):
    deg = degp_b[slot, 0, :, 0] + degp_b[slot, 1, :, 0]
    return jnp.where(deg > 0.0, lax.rsqrt(deg), 1.0)


def _tc_first(x, degp, W0):
    def body(x_b, degp_b, w_ref, p0_b, p1_b, p2_b, norm_b):
        ns = [_norm_from(degp_b, r) for r in range(R)]
        nd = [_norm_from(degp_b, R + r) for r in range(R)]
        one = jnp.ones((BN,), jnp.float32)
        norm_b[...] = jnp.stack(
            [ns[0], ns[1], ns[2], one, nd[0], nd[1], nd[2], one], axis=-1)
        xb = x_b[...]
        for r, p_b in enumerate([p0_b, p1_b, p2_b]):
            p_b[...] = ns[r][:, None] * _dot(xb, w_ref[r])

    out_shape = (
        [jax.ShapeDtypeStruct((N, D), jnp.float32)] * 3
        + [jax.ShapeDtypeStruct((N, 8), jnp.float32)]
    )
    return pl.pallas_call(
        body,
        grid=(NBLK,),
        in_specs=[
            pl.BlockSpec((BN, D), lambda i: (i, 0)),
            pl.BlockSpec((6, NC, BN, D), lambda i: (0, 0, i, 0)),
            pl.BlockSpec((R, D, D), lambda i: (0, 0, 0)),
        ],
        out_specs=[
            pl.BlockSpec((BN, D), lambda i: (i, 0)),
            pl.BlockSpec((BN, D), lambda i: (i, 0)),
            pl.BlockSpec((BN, D), lambda i: (i, 0)),
            pl.BlockSpec((BN, 8), lambda i: (i, 0)),
        ],
        out_shape=out_shape,
    )(x, degp, W0)


def _tc_mid(ap, norms, Wl, b_prev, relu, matmul):
    def body(ap_b, norm_b, w_ref, b_ref, *outs):
        nrm = norm_b[...]
        h = jnp.zeros((BN, D), jnp.float32)
        for r in range(R):
            h = h + nrm[:, 4 + r][:, None] * (ap_b[r, 0] + ap_b[r, 1])
        h = h + (b_ref[0] + b_ref[1] + b_ref[2])[None, :]
        if relu:
            h = jnp.maximum(h, 0.0)
        if matmul:
            for r, p_b in enumerate(outs):
                p_b[...] = nrm[:, r][:, None] * _dot(h, w_ref[r])
        else:
            outs[0][...] = h

    n_out = 3 if matmul else 1
    return pl.pallas_call(
        body,
        grid=(NBLK,),
        in_specs=[
            pl.BlockSpec((R, NC, BN, D), lambda i: (0, 0, i, 0)),
            pl.BlockSpec((BN, 8), lambda i: (i, 0)),
            pl.BlockSpec((R, D, D), lambda i: (0, 0, 0)),
            pl.BlockSpec((R, D), lambda i: (0, 0)),
        ],
        out_specs=[pl.BlockSpec((BN, D), lambda i: (i, 0))] * n_out,
        out_shape=[jax.ShapeDtypeStruct((N, D), jnp.float32)] * n_out,
    )(ap, norms, Wl, b_prev)


def _tc_final_sum(ap):
    def body(ap_b, o_b):
        o_b[...] = ap_b[0, 0] + ap_b[0, 1]

    return pl.pallas_call(
        body,
        grid=(NBLK,),
        in_specs=[pl.BlockSpec((1, NC, BN, D), lambda i: (0, 0, i, 0))],
        out_specs=pl.BlockSpec((BN, D), lambda i: (i, 0)),
        out_shape=jax.ShapeDtypeStruct((N, D), jnp.float32),
    )(ap)


def kernel(x, Ws, bs, edge_index_r0, edge_index_r1, edge_index_r2):
    e0 = edge_index_r0.astype(jnp.int32)
    e1 = edge_index_r1.astype(jnp.int32)
    e2 = edge_index_r2.astype(jnp.int32)
    s0, d0 = e0[0], e0[1]
    s1, d1 = e1[0], e1[1]
    s2, d2 = e2[0], e2[1]

    degp = _sc_degrees(s0, s1, s2, d0, d1, d2)
    p0, p1, p2, norms = _tc_first(x, degp, Ws[0])

    for layer in range(1, L):
        ap = _sc_spmm(p0, p1, p2, s0, d0, s1, d1, s2, d2)
        p0, p1, p2 = _tc_mid(ap, norms, Ws[layer], bs[layer - 1],
                             relu=True, matmul=True)

    ap = _sc_spmm(p0, p1, p2, s0, d0, s1, d1, s2, d2)
    hL = _tc_mid(ap, norms, Ws[0], bs[L - 1], relu=False, matmul=False)[0]

    ap2 = _sc_spmm(hL, hL, hL, s0, d0, s1, d1, s2, d2, shared=True)
    return _tc_final_sum(ap2)


# speedup vs baseline: 6.2484x; 1.1426x over previous
import functools

import jax
import jax.numpy as jnp
from jax import lax
from jax.experimental import pallas as pl
from jax.experimental.pallas import tpu as pltpu
from jax.experimental.pallas import tpu_sc as plsc

N = 10000
D = 128
R = 3
L = 4
E = 160000

NC = 2
NS = 16
NW = NC * NS
CHUNK = 128
NCHUNKS = E // CHUNK
TRIPS_FLOOR = NCHUNKS // NW
TRIPS_EXTRA = NCHUNKS % NW
ROWS_PER_TILE = 624
ROWS_TAIL = N - NS * ROWS_PER_TILE
BN = 1000
NBLK = N // BN

_MESH = plsc.VectorSubcoreMesh(core_axis_name="c", subcore_axis_name="s")

def _deg_pass(e_h, acc, ones, wid, didxA, didxB, iA, iB, sA, sB):
    T = TRIPS_FLOOR

    def idx_start(t, buf, sem):
        off = (t * NW + wid) * CHUNK
        pltpu.make_async_copy(e_h.at[pl.ds(off, CHUNK)], buf, sem).start()

    def idx_wait(buf, sem):
        pltpu.make_async_copy(e_h.at[pl.ds(0, CHUNK)], buf, sem).wait()

    def scatter_start(d_buf, sem):
        pltpu.make_async_copy(ones, acc.at[d_buf], sem).start(add=True)

    def scatter_wait(d_buf, sem):
        pltpu.make_async_copy(ones, acc.at[d_buf], sem).wait()

    pltpu.sync_copy(e_h.at[pl.ds(wid * CHUNK, CHUNK)], didxA)
    pltpu.sync_copy(ones, acc.at[didxA], add=True)
    pltpu.sync_copy(e_h.at[pl.ds((NW + wid) * CHUNK, CHUNK)], didxA)
    idx_start(2, didxB, iB)

    @pl.loop(0, (T - 1) // 2)
    def _(g):
        scatter_start(didxA, sA)
        idx_wait(didxB, iB)
        scatter_start(didxB, sB)
        scatter_wait(didxA, sA)
        idx_start(jnp.minimum(2 * g + 3, T - 1), didxA, iA)
        scatter_wait(didxB, sB)
        idx_start(jnp.minimum(2 * g + 4, T - 1), didxB, iB)
        idx_wait(didxA, iA)

    idx_wait(didxB, iB)

    @pl.when(wid < TRIPS_EXTRA)
    def _():
        pltpu.sync_copy(e_h.at[pl.ds((T * NW + wid) * CHUNK, CHUNK)], didxA)
        pltpu.sync_copy(ones, acc.at[didxA], add=True)


def _sc_degrees(s0, s1, s2, d0, d1, d2):
    zeros = jnp.zeros((N, D), jnp.float32)

    @functools.partial(
        pl.kernel,
        out_type=jax.ShapeDtypeStruct((6, NC, N, D), jnp.float32),
        mesh=_MESH,
        scratch_types=[
            pltpu.VMEM((CHUNK,), jnp.int32),
            pltpu.VMEM((CHUNK,), jnp.int32),
            pltpu.VMEM((CHUNK, D), jnp.float32),
            pltpu.VMEM_SHARED((N, D), jnp.float32),
            pltpu.SemaphoreType.DMA,
            pltpu.SemaphoreType.DMA,
            pltpu.SemaphoreType.DMA,
            pltpu.SemaphoreType.DMA,
        ],
    )
    def k(z_h, e0_h, e1_h, e2_h, e3_h, e4_h, e5_h, out_h,
          didxA, didxB, ones, acc, iA, iB, sA, sB):
        core = lax.axis_index("c")
        sub = lax.axis_index("s")
        wid = core * NS + sub
        row0 = sub * ROWS_PER_TILE

        one = jnp.ones((16,), jnp.float32)

        @pl.loop(0, CHUNK)
        def _(i):
            for j in range(D // 16):
                ones[i, pl.ds(j * 16, 16)] = one

        for slot, e_h in enumerate([e0_h, e1_h, e2_h, e3_h, e4_h, e5_h]):
            pltpu.sync_copy(z_h.at[pl.ds(row0, ROWS_PER_TILE)],
                            acc.at[pl.ds(row0, ROWS_PER_TILE)])

            @pl.when(sub == NS - 1)
            def _():
                pltpu.sync_copy(z_h.at[pl.ds(N - ROWS_TAIL, ROWS_TAIL)],
                                acc.at[pl.ds(N - ROWS_TAIL, ROWS_TAIL)])

            plsc.subcore_barrier()
            _deg_pass(e_h, acc, ones, wid, didxA, didxB, iA, iB, sA, sB)
            plsc.subcore_barrier()
            pltpu.sync_copy(acc.at[pl.ds(row0, ROWS_PER_TILE)],
                            out_h.at[slot, core, pl.ds(row0, ROWS_PER_TILE)])

            @pl.when(sub == NS - 1)
            def _():
                pltpu.sync_copy(
                    acc.at[pl.ds(N - ROWS_TAIL, ROWS_TAIL)],
                    out_h.at[slot, core, pl.ds(N - ROWS_TAIL, ROWS_TAIL)])

            plsc.subcore_barrier()

    return k(zeros, s0, s1, s2, d0, d1, d2)


def _edge_pass(p_h, s_h, d_h, acc, wid, slots):
    T = TRIPS_FLOOR
    (sidxA, didxA, rowsA, gA, iA, sA), \
        (sidxB, didxB, rowsB, gB, iB, sB), \
        (sidxC, didxC, rowsC, gC, iC, sC) = slots

    def idx_start(t, s_buf, d_buf, sem):
        off = (t * NW + wid) * CHUNK
        pltpu.make_async_copy(s_h.at[pl.ds(off, CHUNK)], s_buf, sem).start()
        pltpu.make_async_copy(d_h.at[pl.ds(off, CHUNK)], d_buf, sem).start()

    def idx_wait(s_buf, d_buf, sem):
        pltpu.make_async_copy(s_h.at[pl.ds(0, CHUNK)], s_buf, sem).wait()
        pltpu.make_async_copy(d_h.at[pl.ds(0, CHUNK)], d_buf, sem).wait()

    def gather_start(s_buf, rows, sem):
        pltpu.make_async_copy(p_h.at[s_buf], rows, sem).start()

    def gather_wait(s_buf, rows, sem):
        pltpu.make_async_copy(p_h.at[s_buf], rows, sem).wait()

    def scatter_start(rows, d_buf, sem):
        pltpu.make_async_copy(rows, acc.at[d_buf], sem).start(add=True)

    def scatter_wait(rows, d_buf, sem):
        pltpu.make_async_copy(rows, acc.at[d_buf], sem).wait()

    pltpu.sync_copy(s_h.at[pl.ds(wid * CHUNK, CHUNK)], sidxA)
    pltpu.sync_copy(d_h.at[pl.ds(wid * CHUNK, CHUNK)], didxA)
    gather_start(sidxA, rowsA, gA)
    pltpu.sync_copy(s_h.at[pl.ds((NW + wid) * CHUNK, CHUNK)], sidxB)
    pltpu.sync_copy(d_h.at[pl.ds((NW + wid) * CHUNK, CHUNK)], didxB)
    gather_start(sidxB, rowsB, gB)
    idx_start(2, sidxC, didxC, iC)

    @pl.loop(0, T // 3)
    def _(g):
        gather_wait(sidxA, rowsA, gA)
        scatter_start(rowsA, didxA, sA)
        idx_wait(sidxC, didxC, iC)
        gather_start(sidxC, rowsC, gC)
        gather_wait(sidxB, rowsB, gB)
        scatter_start(rowsB, didxB, sB)
        scatter_wait(rowsA, didxA, sA)
        idx_start(jnp.minimum(3 * g + 3, T - 1), sidxA, didxA, iA)
        idx_wait(sidxA, didxA, iA)
        gather_start(sidxA, rowsA, gA)
        gather_wait(sidxC, rowsC, gC)
        scatter_start(rowsC, didxC, sC)
        scatter_wait(rowsB, didxB, sB)
        idx_start(jnp.minimum(3 * g + 4, T - 1), sidxB, didxB, iB)
        idx_wait(sidxB, didxB, iB)
        gather_start(sidxB, rowsB, gB)
        scatter_wait(rowsC, didxC, sC)
        idx_start(jnp.minimum(3 * g + 5, T - 1), sidxC, didxC, iC)

    gather_wait(sidxA, rowsA, gA)
    gather_wait(sidxB, rowsB, gB)
    idx_wait(sidxC, didxC, iC)

    @pl.when(wid < TRIPS_EXTRA)
    def _():
        offx = (T * NW + wid) * CHUNK
        pltpu.sync_copy(s_h.at[pl.ds(offx, CHUNK)], sidxA)
        pltpu.sync_copy(d_h.at[pl.ds(offx, CHUNK)], didxA)
        gather_start(sidxA, rowsA, gA)
        gather_wait(sidxA, rowsA, gA)
        pltpu.sync_copy(rowsA, acc.at[didxA], add=True)


def _sc_spmm(p0, p1, p2, s0, d0, s1, d1, s2, d2, shared=False):
    n_out = 1 if shared else R
    out_t = jax.ShapeDtypeStruct((n_out, NC, N, D), jnp.float32)

    idx_t = pltpu.VMEM((CHUNK,), jnp.int32)
    rows_t = pltpu.VMEM((CHUNK, D), jnp.float32)
    sem_t = pltpu.SemaphoreType.DMA

    @functools.partial(
        pl.kernel,
        out_type=out_t,
        mesh=_MESH,
        scratch_types=(
            [idx_t] * 6 + [rows_t] * 3
            + [pltpu.VMEM_SHARED((N, D), jnp.float32)] + [sem_t] * 9
        ),
    )
    def k(z_h, p0_h, p1_h, p2_h, s0_h, d0_h, s1_h, d1_h, s2_h, d2_h, out_h,
          sidxA, didxA, sidxB, didxB, sidxC, didxC, rowsA, rowsB, rowsC, acc,
          gA, gB, gC, iA, iB, iC, sA, sB, sC):
        core = lax.axis_index("c")
        sub = lax.axis_index("s")
        wid = core * NS + sub
        row0 = sub * ROWS_PER_TILE
        slots = ((sidxA, didxA, rowsA, gA, iA, sA),
                 (sidxB, didxB, rowsB, gB, iB, sB),
                 (sidxC, didxC, rowsC, gC, iC, sC))

        def zero_acc():
            pltpu.sync_copy(z_h.at[pl.ds(row0, ROWS_PER_TILE)],
                            acc.at[pl.ds(row0, ROWS_PER_TILE)])

            @pl.when(sub == NS - 1)
            def _():
                pltpu.sync_copy(z_h.at[pl.ds(N - ROWS_TAIL, ROWS_TAIL)],
                                acc.at[pl.ds(N - ROWS_TAIL, ROWS_TAIL)])

        def drain(slot):
            pltpu.sync_copy(acc.at[pl.ds(row0, ROWS_PER_TILE)],
                            out_h.at[slot, core, pl.ds(row0, ROWS_PER_TILE)])

            @pl.when(sub == NS - 1)
            def _():
                pltpu.sync_copy(
                    acc.at[pl.ds(N - ROWS_TAIL, ROWS_TAIL)],
                    out_h.at[slot, core, pl.ds(N - ROWS_TAIL, ROWS_TAIL)])

        triples = [(p0_h, s0_h, d0_h), (p1_h, s1_h, d1_h), (p2_h, s2_h, d2_h)]
        if shared:
            zero_acc()
            plsc.subcore_barrier()
            for p_h, s_h, d_h in triples:
                _edge_pass(p_h, s_h, d_h, acc, wid, slots)
            plsc.subcore_barrier()
            drain(0)
        else:
            for r, (p_h, s_h, d_h) in enumerate(triples):
                zero_acc()
                plsc.subcore_barrier()
                _edge_pass(p_h, s_h, d_h, acc, wid, slots)
                plsc.subcore_barrier()
                drain(r)
                plsc.subcore_barrier()

    zeros = jnp.zeros((N, D), jnp.float32)
    return k(zeros, p0, p1, p2, s0, d0, s1, d1, s2, d2)


def _dot(a, b):
    return jax.lax.dot_general(
        a, b, (((1,), (0,)), ((), ())),
        precision=jax.lax.Precision.HIGHEST,
        preferred_element_type=jnp.float32,
    )


def _norm_from(degp_b, slot):
    deg = degp_b[slot, 0, :, 0] + degp_b[slot, 1, :, 0]
    return jnp.where(deg > 0.0, lax.rsqrt(deg), 1.0)


def _tc_first(x, degp, W0):
    def body(x_b, degp_b, w_ref, p0_b, p1_b, p2_b, norm_b):
        ns = [_norm_from(degp_b, r) for r in range(R)]
        nd = [_norm_from(degp_b, R + r) for r in range(R)]
        one = jnp.ones((BN,), jnp.float32)
        norm_b[...] = jnp.stack(
            [ns[0], ns[1], ns[2], one, nd[0], nd[1], nd[2], one], axis=-1)
        xb = x_b[...]
        for r, p_b in enumerate([p0_b, p1_b, p2_b]):
            p_b[...] = ns[r][:, None] * _dot(xb, w_ref[r])

    out_shape = (
        [jax.ShapeDtypeStruct((N, D), jnp.float32)] * 3
        + [jax.ShapeDtypeStruct((N, 8), jnp.float32)]
    )
    return pl.pallas_call(
        body,
        grid=(NBLK,),
        in_specs=[
            pl.BlockSpec((BN, D), lambda i: (i, 0)),
            pl.BlockSpec((6, NC, BN, D), lambda i: (0, 0, i, 0)),
            pl.BlockSpec((R, D, D), lambda i: (0, 0, 0)),
        ],
        out_specs=[
            pl.BlockSpec((BN, D), lambda i: (i, 0)),
            pl.BlockSpec((BN, D), lambda i: (i, 0)),
            pl.BlockSpec((BN, D), lambda i: (i, 0)),
            pl.BlockSpec((BN, 8), lambda i: (i, 0)),
        ],
        out_shape=out_shape,
    )(x, degp, W0)


def _tc_mid(ap, norms, Wl, b_prev, relu, matmul):
    def body(ap_b, norm_b, w_ref, b_ref, *outs):
        nrm = norm_b[...]
        h = jnp.zeros((BN, D), jnp.float32)
        for r in range(R):
            h = h + nrm[:, 4 + r][:, None] * (ap_b[r, 0] + ap_b[r, 1])
        h = h + (b_ref[0] + b_ref[1] + b_ref[2])[None, :]
        if relu:
            h = jnp.maximum(h, 0.0)
        if matmul:
            for r, p_b in enumerate(outs):
                p_b[...] = nrm[:, r][:, None] * _dot(h, w_ref[r])
        else:
            outs[0][...] = h

    n_out = 3 if matmul else 1
    return pl.pallas_call(
        body,
        grid=(NBLK,),
        in_specs=[
            pl.BlockSpec((R, NC, BN, D), lambda i: (0, 0, i, 0)),
            pl.BlockSpec((BN, 8), lambda i: (i, 0)),
            pl.BlockSpec((R, D, D), lambda i: (0, 0, 0)),
            pl.BlockSpec((R, D), lambda i: (0, 0)),
        ],
        out_specs=[pl.BlockSpec((BN, D), lambda i: (i, 0))] * n_out,
        out_shape=[jax.ShapeDtypeStruct((N, D), jnp.float32)] * n_out,
    )(ap, norms, Wl, b_prev)


def _tc_final_sum(ap):
    def body(ap_b, o_b):
        o_b[...] = ap_b[0, 0] + ap_b[0, 1]

    return pl.pallas_call(
        body,
        grid=(NBLK,),
        in_specs=[pl.BlockSpec((1, NC, BN, D), lambda i: (0, 0, i, 0))],
        out_specs=pl.BlockSpec((BN, D), lambda i: (i, 0)),
        out_shape=jax.ShapeDtypeStruct((N, D), jnp.float32),
    )(ap)


def kernel(x, Ws, bs, edge_index_r0, edge_index_r1, edge_index_r2):
    e0 = edge_index_r0.astype(jnp.int32)
    e1 = edge_index_r1.astype(jnp.int32)
    e2 = edge_index_r2.astype(jnp.int32)
    s0, d0 = e0[0], e0[1]
    s1, d1 = e1[0], e1[1]
    s2, d2 = e2[0], e2[1]

    degp = _sc_degrees(s0, s1, s2, d0, d1, d2)
    p0, p1, p2, norms = _tc_first(x, degp, Ws[0])

    for layer in range(1, L):
        ap = _sc_spmm(p0, p1, p2, s0, d0, s1, d1, s2, d2)
        p0, p1, p2 = _tc_mid(ap, norms, Ws[layer], bs[layer - 1],
                             relu=True, matmul=True)

    ap = _sc_spmm(p0, p1, p2, s0, d0, s1, d1, s2, d2)
    hL = _tc_mid(ap, norms, Ws[0], bs[L - 1], relu=False, matmul=False)[0]

    ap2 = _sc_spmm(hL, hL, hL, s0, d0, s1, d1, s2, d2, shared=True)
    return _tc_final_sum(ap2)


# speedup vs baseline: 6.5595x; 1.0498x over previous
import functools

import jax
import jax.numpy as jnp
from jax import lax
from jax.experimental import pallas as pl
from jax.experimental.pallas import tpu as pltpu
from jax.experimental.pallas import tpu_sc as plsc

N = 10000
D = 128
R = 3
L = 4
E = 160000

NC = 2
NS = 16
NW = NC * NS
CHUNK = 128
NCHUNKS = E // CHUNK
TRIPS_FLOOR = NCHUNKS // NW
TRIPS_EXTRA = NCHUNKS % NW
ROWS_PER_TILE = 624
ROWS_TAIL = N - NS * ROWS_PER_TILE
BN = 1000
NBLK = N // BN

_MESH = plsc.VectorSubcoreMesh(core_axis_name="c", subcore_axis_name="s")

def _deg_pass(e_h, acc, ones, wid, didxA, didxB, iA, iB, sA, sB):
    T = TRIPS_FLOOR

    def idx_start(t, buf, sem):
        off = (t * NW + wid) * CHUNK
        pltpu.make_async_copy(e_h.at[pl.ds(off, CHUNK)], buf, sem).start()

    def idx_wait(buf, sem):
        pltpu.make_async_copy(e_h.at[pl.ds(0, CHUNK)], buf, sem).wait()

    def scatter_start(d_buf, sem):
        pltpu.make_async_copy(ones, acc.at[d_buf], sem).start(add=True)

    def scatter_wait(d_buf, sem):
        pltpu.make_async_copy(ones, acc.at[d_buf], sem).wait()

    pltpu.sync_copy(e_h.at[pl.ds(wid * CHUNK, CHUNK)], didxA)
    pltpu.sync_copy(ones, acc.at[didxA], add=True)
    pltpu.sync_copy(e_h.at[pl.ds((NW + wid) * CHUNK, CHUNK)], didxA)
    idx_start(2, didxB, iB)

    @pl.loop(0, (T - 1) // 2)
    def _(g):
        scatter_start(didxA, sA)
        idx_wait(didxB, iB)
        scatter_start(didxB, sB)
        scatter_wait(didxA, sA)
        idx_start(jnp.minimum(2 * g + 3, T - 1), didxA, iA)
        scatter_wait(didxB, sB)
        idx_start(jnp.minimum(2 * g + 4, T - 1), didxB, iB)
        idx_wait(didxA, iA)

    idx_wait(didxB, iB)

    @pl.when(wid < TRIPS_EXTRA)
    def _():
        pltpu.sync_copy(e_h.at[pl.ds((T * NW + wid) * CHUNK, CHUNK)], didxA)
        pltpu.sync_copy(ones, acc.at[didxA], add=True)


def _sc_degrees(s0, s1, s2, d0, d1, d2):
    zeros = jnp.zeros((N, D), jnp.float32)

    @functools.partial(
        pl.kernel,
        out_type=jax.ShapeDtypeStruct((6, NC, N, D), jnp.float32),
        mesh=_MESH,
        scratch_types=[
            pltpu.VMEM((CHUNK,), jnp.int32),
            pltpu.VMEM((CHUNK,), jnp.int32),
            pltpu.VMEM((CHUNK, D), jnp.float32),
            pltpu.VMEM_SHARED((N, D), jnp.float32),
            pltpu.SemaphoreType.DMA,
            pltpu.SemaphoreType.DMA,
            pltpu.SemaphoreType.DMA,
            pltpu.SemaphoreType.DMA,
        ],
    )
    def k(z_h, e0_h, e1_h, e2_h, e3_h, e4_h, e5_h, out_h,
          didxA, didxB, ones, acc, iA, iB, sA, sB):
        core = lax.axis_index("c")
        sub = lax.axis_index("s")
        wid = core * NS + sub
        row0 = sub * ROWS_PER_TILE

        one = jnp.ones((16,), jnp.float32)

        @pl.loop(0, CHUNK)
        def _(i):
            for j in range(D // 16):
                ones[i, pl.ds(j * 16, 16)] = one

        for slot, e_h in enumerate([e0_h, e1_h, e2_h, e3_h, e4_h, e5_h]):
            pltpu.sync_copy(z_h.at[pl.ds(row0, ROWS_PER_TILE)],
                            acc.at[pl.ds(row0, ROWS_PER_TILE)])

            @pl.when(sub == NS - 1)
            def _():
                pltpu.sync_copy(z_h.at[pl.ds(N - ROWS_TAIL, ROWS_TAIL)],
                                acc.at[pl.ds(N - ROWS_TAIL, ROWS_TAIL)])

            plsc.subcore_barrier()
            _deg_pass(e_h, acc, ones, wid, didxA, didxB, iA, iB, sA, sB)
            plsc.subcore_barrier()
            pltpu.sync_copy(acc.at[pl.ds(row0, ROWS_PER_TILE)],
                            out_h.at[slot, core, pl.ds(row0, ROWS_PER_TILE)])

            @pl.when(sub == NS - 1)
            def _():
                pltpu.sync_copy(
                    acc.at[pl.ds(N - ROWS_TAIL, ROWS_TAIL)],
                    out_h.at[slot, core, pl.ds(N - ROWS_TAIL, ROWS_TAIL)])

            plsc.subcore_barrier()

    return k(zeros, s0, s1, s2, d0, d1, d2)


def _edge_pass(p_h, s_h, d_h, acc, wid, slots):
    T = TRIPS_FLOOR
    (sidxA, didxA, rowsA, gA, iAs, iAd, sA), \
        (sidxB, didxB, rowsB, gB, iBs, iBd, sB), \
        (sidxC, didxC, rowsC, gC, iCs, iCd, sC) = slots

    def sidx_start(t, buf, sem):
        off = (t * NW + wid) * CHUNK
        pltpu.make_async_copy(s_h.at[pl.ds(off, CHUNK)], buf, sem).start()

    def sidx_wait(buf, sem):
        pltpu.make_async_copy(s_h.at[pl.ds(0, CHUNK)], buf, sem).wait()

    def didx_start(t, buf, sem):
        off = (t * NW + wid) * CHUNK
        pltpu.make_async_copy(d_h.at[pl.ds(off, CHUNK)], buf, sem).start()

    def didx_wait(buf, sem):
        pltpu.make_async_copy(d_h.at[pl.ds(0, CHUNK)], buf, sem).wait()

    def gather_start(s_buf, rows, sem):
        pltpu.make_async_copy(p_h.at[s_buf], rows, sem).start()

    def gather_wait(s_buf, rows, sem):
        pltpu.make_async_copy(p_h.at[s_buf], rows, sem).wait()

    def scatter_start(rows, d_buf, sem):
        pltpu.make_async_copy(rows, acc.at[d_buf], sem).start(add=True)

    def scatter_wait(rows, d_buf, sem):
        pltpu.make_async_copy(rows, acc.at[d_buf], sem).wait()

    pltpu.sync_copy(s_h.at[pl.ds(wid * CHUNK, CHUNK)], sidxA)
    didx_start(0, didxA, iAd)
    gather_start(sidxA, rowsA, gA)
    pltpu.sync_copy(s_h.at[pl.ds((NW + wid) * CHUNK, CHUNK)], sidxB)
    didx_start(1, didxB, iBd)
    gather_start(sidxB, rowsB, gB)
    sidx_start(2, sidxC, iCs)
    didx_start(2, didxC, iCd)

    @pl.loop(0, T // 3)
    def _(g):
        uA = jnp.minimum(3 * g + 3, T - 1)
        uB = jnp.minimum(3 * g + 4, T - 1)
        uC = jnp.minimum(3 * g + 5, T - 1)
        gather_wait(sidxA, rowsA, gA)
        didx_wait(didxA, iAd)
        scatter_start(rowsA, didxA, sA)
        sidx_start(uA, sidxA, iAs)
        sidx_wait(sidxC, iCs)
        gather_start(sidxC, rowsC, gC)
        gather_wait(sidxB, rowsB, gB)
        didx_wait(didxB, iBd)
        scatter_start(rowsB, didxB, sB)
        sidx_start(uB, sidxB, iBs)
        scatter_wait(rowsA, didxA, sA)
        didx_start(uA, didxA, iAd)
        sidx_wait(sidxA, iAs)
        gather_start(sidxA, rowsA, gA)
        gather_wait(sidxC, rowsC, gC)
        didx_wait(didxC, iCd)
        scatter_start(rowsC, didxC, sC)
        sidx_start(uC, sidxC, iCs)
        scatter_wait(rowsB, didxB, sB)
        didx_start(uB, didxB, iBd)
        sidx_wait(sidxB, iBs)
        gather_start(sidxB, rowsB, gB)
        scatter_wait(rowsC, didxC, sC)
        didx_start(uC, didxC, iCd)

    gather_wait(sidxA, rowsA, gA)
    gather_wait(sidxB, rowsB, gB)
    didx_wait(didxA, iAd)
    didx_wait(didxB, iBd)
    sidx_wait(sidxC, iCs)
    didx_wait(didxC, iCd)

    @pl.when(wid < TRIPS_EXTRA)
    def _():
        offx = (T * NW + wid) * CHUNK
        pltpu.sync_copy(s_h.at[pl.ds(offx, CHUNK)], sidxA)
        pltpu.sync_copy(d_h.at[pl.ds(offx, CHUNK)], didxA)
        gather_start(sidxA, rowsA, gA)
        gather_wait(sidxA, rowsA, gA)
        pltpu.sync_copy(rowsA, acc.at[didxA], add=True)


def _sc_spmm(p0, p1, p2, s0, d0, s1, d1, s2, d2, shared=False):
    n_out = 1 if shared else R
    out_t = jax.ShapeDtypeStruct((n_out, NC, N, D), jnp.float32)

    idx_t = pltpu.VMEM((CHUNK,), jnp.int32)
    rows_t = pltpu.VMEM((CHUNK, D), jnp.float32)
    sem_t = pltpu.SemaphoreType.DMA

    @functools.partial(
        pl.kernel,
        out_type=out_t,
        mesh=_MESH,
        scratch_types=(
            [idx_t] * 6 + [rows_t] * 3
            + [pltpu.VMEM_SHARED((N, D), jnp.float32)] + [sem_t] * 12
        ),
    )
    def k(z_h, p0_h, p1_h, p2_h, s0_h, d0_h, s1_h, d1_h, s2_h, d2_h, out_h,
          sidxA, didxA, sidxB, didxB, sidxC, didxC, rowsA, rowsB, rowsC, acc,
          gA, gB, gC, iAs, iAd, iBs, iBd, iCs, iCd, sA, sB, sC):
        core = lax.axis_index("c")
        sub = lax.axis_index("s")
        wid = core * NS + sub
        row0 = sub * ROWS_PER_TILE
        slots = ((sidxA, didxA, rowsA, gA, iAs, iAd, sA),
                 (sidxB, didxB, rowsB, gB, iBs, iBd, sB),
                 (sidxC, didxC, rowsC, gC, iCs, iCd, sC))

        def zero_acc():
            pltpu.sync_copy(z_h.at[pl.ds(row0, ROWS_PER_TILE)],
                            acc.at[pl.ds(row0, ROWS_PER_TILE)])

            @pl.when(sub == NS - 1)
            def _():
                pltpu.sync_copy(z_h.at[pl.ds(N - ROWS_TAIL, ROWS_TAIL)],
                                acc.at[pl.ds(N - ROWS_TAIL, ROWS_TAIL)])

        def drain(slot):
            pltpu.sync_copy(acc.at[pl.ds(row0, ROWS_PER_TILE)],
                            out_h.at[slot, core, pl.ds(row0, ROWS_PER_TILE)])

            @pl.when(sub == NS - 1)
            def _():
                pltpu.sync_copy(
                    acc.at[pl.ds(N - ROWS_TAIL, ROWS_TAIL)],
                    out_h.at[slot, core, pl.ds(N - ROWS_TAIL, ROWS_TAIL)])

        triples = [(p0_h, s0_h, d0_h), (p1_h, s1_h, d1_h), (p2_h, s2_h, d2_h)]
        if shared:
            zero_acc()
            plsc.subcore_barrier()
            for p_h, s_h, d_h in triples:
                _edge_pass(p_h, s_h, d_h, acc, wid, slots)
            plsc.subcore_barrier()
            drain(0)
        else:
            for r, (p_h, s_h, d_h) in enumerate(triples):
                zero_acc()
                plsc.subcore_barrier()
                _edge_pass(p_h, s_h, d_h, acc, wid, slots)
                plsc.subcore_barrier()
                drain(r)
                plsc.subcore_barrier()

    zeros = jnp.zeros((N, D), jnp.float32)
    return k(zeros, p0, p1, p2, s0, d0, s1, d1, s2, d2)


def _dot(a, b):
    return jax.lax.dot_general(
        a, b, (((1,), (0,)), ((), ())),
        precision=jax.lax.Precision.HIGHEST,
        preferred_element_type=jnp.float32,
    )


def _norm_from(degp_b, slot):
    deg = degp_b[slot, 0, :, 0] + degp_b[slot, 1, :, 0]
    return jnp.where(deg > 0.0, lax.rsqrt(deg), 1.0)


def _tc_first(x, degp, W0):
    def body(x_b, degp_b, w_ref, p0_b, p1_b, p2_b, norm_b):
        ns = [_norm_from(degp_b, r) for r in range(R)]
        nd = [_norm_from(degp_b, R + r) for r in range(R)]
        one = jnp.ones((BN,), jnp.float32)
        norm_b[...] = jnp.stack(
            [ns[0], ns[1], ns[2], one, nd[0], nd[1], nd[2], one], axis=-1)
        xb = x_b[...]
        for r, p_b in enumerate([p0_b, p1_b, p2_b]):
            p_b[...] = ns[r][:, None] * _dot(xb, w_ref[r])

    out_shape = (
        [jax.ShapeDtypeStruct((N, D), jnp.float32)] * 3
        + [jax.ShapeDtypeStruct((N, 8), jnp.float32)]
    )
    return pl.pallas_call(
        body,
        grid=(NBLK,),
        in_specs=[
            pl.BlockSpec((BN, D), lambda i: (i, 0)),
            pl.BlockSpec((6, NC, BN, D), lambda i: (0, 0, i, 0)),
            pl.BlockSpec((R, D, D), lambda i: (0, 0, 0)),
        ],
        out_specs=[
            pl.BlockSpec((BN, D), lambda i: (i, 0)),
            pl.BlockSpec((BN, D), lambda i: (i, 0)),
            pl.BlockSpec((BN, D), lambda i: (i, 0)),
            pl.BlockSpec((BN, 8), lambda i: (i, 0)),
        ],
        out_shape=out_shape,
    )(x, degp, W0)


def _tc_mid(ap, norms, Wl, b_prev, relu, matmul):
    def body(ap_b, norm_b, w_ref, b_ref, *outs):
        nrm = norm_b[...]
        h = jnp.zeros((BN, D), jnp.float32)
        for r in range(R):
            h = h + nrm[:, 4 + r][:, None] * (ap_b[r, 0] + ap_b[r, 1])
        h = h + (b_ref[0] + b_ref[1] + b_ref[2])[None, :]
        if relu:
            h = jnp.maximum(h, 0.0)
        if matmul:
            for r, p_b in enumerate(outs):
                p_b[...] = nrm[:, r][:, None] * _dot(h, w_ref[r])
        else:
            outs[0][...] = h

    n_out = 3 if matmul else 1
    return pl.pallas_call(
        body,
        grid=(NBLK,),
        in_specs=[
            pl.BlockSpec((R, NC, BN, D), lambda i: (0, 0, i, 0)),
            pl.BlockSpec((BN, 8), lambda i: (i, 0)),
            pl.BlockSpec((R, D, D), lambda i: (0, 0, 0)),
            pl.BlockSpec((R, D), lambda i: (0, 0)),
        ],
        out_specs=[pl.BlockSpec((BN, D), lambda i: (i, 0))] * n_out,
        out_shape=[jax.ShapeDtypeStruct((N, D), jnp.float32)] * n_out,
    )(ap, norms, Wl, b_prev)


def _tc_final_sum(ap):
    def body(ap_b, o_b):
        o_b[...] = ap_b[0, 0] + ap_b[0, 1]

    return pl.pallas_call(
        body,
        grid=(NBLK,),
        in_specs=[pl.BlockSpec((1, NC, BN, D), lambda i: (0, 0, i, 0))],
        out_specs=pl.BlockSpec((BN, D), lambda i: (i, 0)),
        out_shape=jax.ShapeDtypeStruct((N, D), jnp.float32),
    )(ap)


def kernel(x, Ws, bs, edge_index_r0, edge_index_r1, edge_index_r2):
    e0 = edge_index_r0.astype(jnp.int32)
    e1 = edge_index_r1.astype(jnp.int32)
    e2 = edge_index_r2.astype(jnp.int32)
    s0, d0 = e0[0], e0[1]
    s1, d1 = e1[0], e1[1]
    s2, d2 = e2[0], e2[1]

    degp = _sc_degrees(s0, s1, s2, d0, d1, d2)
    p0, p1, p2, norms = _tc_first(x, degp, Ws[0])

    for layer in range(1, L):
        ap = _sc_spmm(p0, p1, p2, s0, d0, s1, d1, s2, d2)
        p0, p1, p2 = _tc_mid(ap, norms, Ws[layer], bs[layer - 1],
                             relu=True, matmul=True)

    ap = _sc_spmm(p0, p1, p2, s0, d0, s1, d1, s2, d2)
    hL = _tc_mid(ap, norms, Ws[0], bs[L - 1], relu=False, matmul=False)[0]

    ap2 = _sc_spmm(hL, hL, hL, s0, d0, s1, d1, s2, d2, shared=True)
    return _tc_final_sum(ap2)
